# baseline (speedup 1.0000x reference)
import functools

import jax
import jax.numpy as jnp
from jax import lax
from jax.experimental import pallas as pl
from jax.experimental.pallas import tpu as pltpu
from jax.experimental.pallas import tpu_sc as plsc

NC = 2
NS = 16
R = 8


@functools.lru_cache(maxsize=None)
def _make_kernel(B, S, V, D):
    NW = NC * NS
    rows_per_w = B // NW
    n_blocks = rows_per_w // R
    mesh = plsc.VectorSubcoreMesh(core_axis_name="c", subcore_axis_name="s")

    SP = (S + 7) // 8 * 8
    LP = 128

    @functools.partial(
        pl.kernel,
        out_type=jax.ShapeDtypeStruct((B, SP, LP), jnp.float32),
        mesh=mesh,
        scratch_types=[
            pltpu.VMEM((rows_per_w, LP), jnp.int32),
            pltpu.VMEM((R, SP, D), jnp.float32),
            pltpu.SemaphoreType.DMA((R,)),
        ],
        compiler_params=pltpu.CompilerParams(use_tc_tiling_on_sc=False),
    )
    def k(xp_hbm, table_hbm, out_hbm, idx_v, rows_v, gsem):
        wid = lax.axis_index("s") * NC + lax.axis_index("c")
        row0 = wid * rows_per_w
        pltpu.sync_copy(xp_hbm.at[pl.ds(row0, rows_per_w)], idx_v)

        def gather(j, b):
            return pltpu.make_async_copy(
                table_hbm.at[idx_v.at[j, pl.ds(0, SP)]], rows_v.at[b], gsem.at[b]
            )

        for b in range(R):
            gather(b, b).start()

        def block(o, carry):
            j0 = o * R
            for b in range(R):
                j = j0 + b
                gather(j, b).wait()
                pltpu.sync_copy(
                    rows_v.at[b],
                    out_hbm.at[row0 + j, pl.ds(0, SP), pl.ds(0, D)],
                )
                gather(j + R, b).start()
            return carry

        lax.fori_loop(0, n_blocks - 1, block, 0)

        j0 = (n_blocks - 1) * R
        for b in range(R):
            j = j0 + b
            gather(j, b).wait()
            pltpu.sync_copy(
                rows_v.at[b],
                out_hbm.at[row0 + j, pl.ds(0, SP), pl.ds(0, D)],
            )

    return k


def kernel(x, table):
    B, S = x.shape
    V, D = table.shape
    xp = jnp.pad(x.astype(jnp.int32), ((0, 0), (0, 128 - S)))
    out_p = _make_kernel(B, S, V, D)(xp, table)
    return lax.slice(out_p, (0, 0, 0), (B, S, D))


# speedup vs baseline: 2.5806x; 2.5806x over previous
import functools

import jax
import jax.numpy as jnp
from jax import lax
from jax.experimental import pallas as pl
from jax.experimental.pallas import tpu as pltpu
from jax.experimental.pallas import tpu_sc as plsc

NC = 2
NS = 16
R = 8


@functools.lru_cache(maxsize=None)
def _make_kernel(B, S, V, D):
    NW = NC * NS
    rows_per_w = B // NW
    n_blocks = rows_per_w // R
    mesh = plsc.VectorSubcoreMesh(core_axis_name="c", subcore_axis_name="s")

    SP = (S + 7) // 8 * 8
    LP = 128

    @functools.partial(
        pl.kernel,
        out_type=jax.ShapeDtypeStruct((B, SP, LP), jnp.float32),
        mesh=mesh,
        scratch_types=[
            pltpu.VMEM((rows_per_w, S), jnp.int32),
            pltpu.VMEM((R, S, D), jnp.float32),
            pltpu.SemaphoreType.DMA((R,)),
        ],
        compiler_params=pltpu.CompilerParams(use_tc_tiling_on_sc=False),
    )
    def k(x_hbm, table_hbm, out_hbm, idx_v, rows_v, gsem):
        wid = lax.axis_index("s") * NC + lax.axis_index("c")
        row0 = wid * rows_per_w
        pltpu.sync_copy(x_hbm.at[pl.ds(row0, rows_per_w)], idx_v)

        def gather(j, b):
            return pltpu.make_async_copy(
                table_hbm.at[idx_v.at[j]], rows_v.at[b], gsem.at[b]
            )

        for b in range(R):
            gather(b, b).start()

        def block(o, carry):
            j0 = o * R
            for b in range(R):
                j = j0 + b
                gather(j, b).wait()
                pltpu.sync_copy(
                    rows_v.at[b],
                    out_hbm.at[row0 + j, pl.ds(0, S), pl.ds(0, D)],
                )
                gather(j + R, b).start()
            return carry

        lax.fori_loop(0, n_blocks - 1, block, 0)

        j0 = (n_blocks - 1) * R
        for b in range(R):
            j = j0 + b
            gather(j, b).wait()
            pltpu.sync_copy(
                rows_v.at[b],
                out_hbm.at[row0 + j, pl.ds(0, S), pl.ds(0, D)],
            )

    return k


def kernel(x, table):
    B, S = x.shape
    V, D = table.shape
    out_p = _make_kernel(B, S, V, D)(x.astype(jnp.int32), table)
    return lax.slice(out_p, (0, 0, 0), (B, S, D))
